# baseline (speedup 1.0000x reference)
import functools

import jax
import jax.numpy as jnp
from jax import lax
from jax.experimental import pallas as pl
from jax.experimental.pallas import tpu as pltpu
from jax.experimental.pallas import tpu_sc as plsc

NUM_NODES = 100000
D = 128
B = 16384
NSLICES = 2

_info = plsc.get_sparse_core_info()
_NC, _NS = _info.num_cores, _info.num_subcores
NW = _NC * _NS


def _sc_gather(emb, memory, src, dst):
    bs = src.shape[0]
    b_per_w = bs // NW
    ch = b_per_w // 2
    mesh = plsc.VectorSubcoreMesh(core_axis_name="c", subcore_axis_name="s")

    @functools.partial(
        pl.kernel,
        mesh=mesh,
        out_type=jax.ShapeDtypeStruct((4, bs, D // 2), jnp.int32),
        scratch_types=[
            pltpu.VMEM((b_per_w,), jnp.int32),
            pltpu.VMEM((b_per_w,), jnp.int32),
            pltpu.VMEM((ch, D), jnp.float32),
            pltpu.VMEM((ch, D), jnp.float32),
            pltpu.VMEM((ch, D // 2), jnp.int32),
            pltpu.VMEM((ch, D // 2), jnp.int32),
            pltpu.SemaphoreType.DMA,
            pltpu.SemaphoreType.DMA,
            pltpu.SemaphoreType.DMA,
            pltpu.SemaphoreType.DMA,
        ],
    )
    def gather_kernel(emb_hbm, mem_hbm, src_hbm, dst_hbm, out_hbm,
                      src_v, dst_v, rows0, rows1, brows0, brows1, g0, g1, w0, w1):
        wid = lax.axis_index("s") * _NC + lax.axis_index("c")
        base = wid * b_per_w
        pltpu.sync_copy(src_hbm.at[pl.ds(base, b_per_w)], src_v)
        pltpu.sync_copy(dst_hbm.at[pl.ds(base, b_per_w)], dst_v)
        rows = (rows0, rows1)
        brows = (brows0, brows1)
        gsem = (g0, g1)
        wsem = (w0, w1)
        chunks = []
        for p, (tab, idxv) in enumerate(
                ((emb_hbm, src_v), (emb_hbm, dst_v), (mem_hbm, src_v), (mem_hbm, dst_v))):
            for h in range(2):
                chunks.append((tab, idxv, p, h))
        n = len(chunks)

        def start_gather(k):
            tab, idxv, _, h = chunks[k]
            return pltpu.async_copy(tab.at[idxv.at[pl.ds(h * ch, ch)]],
                                    rows[k % 2], gsem[k % 2])

        def start_write(k):
            _, _, p, h = chunks[k]
            return pltpu.async_copy(brows[k % 2],
                                    out_hbm.at[p, pl.ds(base + h * ch, ch)],
                                    wsem[k % 2])

        def convert(k):
            fbuf = rows[k % 2]
            bbuf = brows[k % 2]

            half = jnp.int32(0x8000)
            himask = jnp.int32(-65536)

            @plsc.parallel_loop(0, ch, step=1, unroll=8)
            def _conv_body(r):
                for q in range(D // 32):
                    a = fbuf[r, pl.ds(q * 32, 16)]
                    b = fbuf[r, pl.ds(q * 32 + 16, 16)]
                    ai = lax.bitcast_convert_type(a, jnp.int32)
                    bi = lax.bitcast_convert_type(b, jnp.int32)
                    lo = lax.shift_right_logical(ai + half, 16)
                    word = lo | ((bi + half) & himask)
                    bbuf[r, pl.ds(q * 16, 16)] = word

        hg = [None] * n
        hw = [None] * n
        hg[0] = start_gather(0)
        for k in range(n):
            hg[k].wait()
            if k + 1 < n:
                hg[k + 1] = start_gather(k + 1)
            if k >= 2:
                hw[k - 2].wait()
            convert(k)
            hw[k] = start_write(k)
        hw[n - 2].wait()
        hw[n - 1].wait()

    return gather_kernel(emb, memory, src, dst)


_BM = 1024


def _mlp_body(g_ref, w1_ref, b1_ref, w2_ref, b2_ref, out_ref):
    acc = jnp.dot(g_ref[0], w1_ref[0].astype(jnp.bfloat16),
                  preferred_element_type=jnp.float32)
    for p in range(1, 4):
        acc += jnp.dot(g_ref[p], w1_ref[p].astype(jnp.bfloat16),
                       preferred_element_type=jnp.float32)
    h1 = jnp.maximum(acc + b1_ref[0][None, :], 0.0)
    out_ref[...] = jnp.sum(h1 * w2_ref[0][None, :], axis=1) + b2_ref[0]


def _tc_mlp(g4, w1r, b1, w2, b2):
    bs = g4.shape[1]
    grid = (bs // _BM,)
    return pl.pallas_call(
        _mlp_body,
        grid=grid,
        in_specs=[
            pl.BlockSpec((4, _BM, D), lambda i: (0, i, 0)),
            pl.BlockSpec((4, D, D), lambda i: (0, 0, 0)),
            pl.BlockSpec((1, D), lambda i: (0, 0)),
            pl.BlockSpec((1, D), lambda i: (0, 0)),
            pl.BlockSpec((1,), lambda i: (0,)),
        ],
        out_specs=pl.BlockSpec((_BM,), lambda i: (i,)),
        out_shape=jax.ShapeDtypeStruct((bs,), jnp.float32),
    )(g4, w1r, b1, w2, b2)


def kernel(src, dst, ts, edge_feat, emb, memory, time_w, time_b, edge_W, edge_b, W1, b1, W2, b2):
    w1r = W1.reshape(D, 4, D).transpose(1, 2, 0)
    c = jnp.arange(D)
    perm = 32 * (c // 32) + (c % 32) // 2 + 16 * (c % 2)
    w1r = w1r[:, perm, :]
    b1r = b1.reshape(1, D)
    w2r = W2.reshape(1, D)
    bs = B // NSLICES
    gs = []
    for s in range(NSLICES):
        sl = slice(s * bs, (s + 1) * bs)
        gi = _sc_gather(emb, memory, src[sl], dst[sl])
        gs.append(lax.bitcast_convert_type(gi, jnp.bfloat16).reshape(4, bs, D))
    outs = [_tc_mlp(g4, w1r, b1r, w2r, b2) for g4 in gs]
    return jnp.concatenate(outs)


# speedup vs baseline: 2.6222x; 2.6222x over previous
import functools

import jax
import jax.numpy as jnp
from jax import lax
from jax.experimental import pallas as pl
from jax.experimental.pallas import tpu as pltpu
from jax.experimental.pallas import tpu_sc as plsc

NUM_NODES = 100000
D = 128
B = 16384

_info = plsc.get_sparse_core_info()
_NC, _NS = _info.num_cores, _info.num_subcores
NW = _NC * _NS
B_PER_W = B // NW
_CH = B_PER_W // 2


def _sc_gather(emb, memory, src, dst):
    mesh = plsc.VectorSubcoreMesh(core_axis_name="c", subcore_axis_name="s")

    @functools.partial(
        pl.kernel,
        mesh=mesh,
        out_type=jax.ShapeDtypeStruct((4, B, D), jnp.float32),
        scratch_types=[
            pltpu.VMEM((B_PER_W,), jnp.int32),
            pltpu.VMEM((B_PER_W,), jnp.int32),
            pltpu.VMEM((_CH, D), jnp.float32),
            pltpu.VMEM((_CH, D), jnp.float32),
            pltpu.SemaphoreType.DMA,
            pltpu.SemaphoreType.DMA,
            pltpu.SemaphoreType.DMA,
            pltpu.SemaphoreType.DMA,
        ],
    )
    def gather_kernel(emb_hbm, mem_hbm, src_hbm, dst_hbm, out_hbm,
                      src_v, dst_v, rows0, rows1, g0, g1, w0, w1):
        wid = lax.axis_index("s") * _NC + lax.axis_index("c")
        base = wid * B_PER_W
        pltpu.sync_copy(src_hbm.at[pl.ds(base, B_PER_W)], src_v)
        pltpu.sync_copy(dst_hbm.at[pl.ds(base, B_PER_W)], dst_v)
        rows = (rows0, rows1)
        gsem = (g0, g1)
        wsem = (w0, w1)
        chunks = []
        for p, (tab, idxv) in enumerate(
                ((emb_hbm, src_v), (emb_hbm, dst_v), (mem_hbm, src_v), (mem_hbm, dst_v))):
            for h in range(2):
                chunks.append((tab, idxv, p, h))
        n = len(chunks)

        def start_gather(k):
            tab, idxv, _, h = chunks[k]
            return pltpu.async_copy(tab.at[idxv.at[pl.ds(h * _CH, _CH)]],
                                    rows[k % 2], gsem[k % 2])

        def start_write(k):
            _, _, p, h = chunks[k]
            return pltpu.async_copy(rows[k % 2],
                                    out_hbm.at[p, pl.ds(base + h * _CH, _CH)],
                                    wsem[k % 2])

        hg = [None] * n
        hw = [None] * n
        hg[0] = start_gather(0)
        for k in range(n):
            hg[k].wait()
            if k + 1 < n:
                if k >= 1:
                    hw[k - 1].wait()
                hg[k + 1] = start_gather(k + 1)
            hw[k] = start_write(k)
        hw[n - 2].wait()
        hw[n - 1].wait()

    return gather_kernel(emb, memory, src, dst)


_BM = 2048


def _mlp_body(g_ref, w1_ref, b1_ref, w2_ref, b2_ref, out_ref):
    acc = jnp.dot(g_ref[0].astype(jnp.bfloat16), w1_ref[0].astype(jnp.bfloat16),
                  preferred_element_type=jnp.float32)
    for p in range(1, 4):
        acc += jnp.dot(g_ref[p].astype(jnp.bfloat16), w1_ref[p].astype(jnp.bfloat16),
                       preferred_element_type=jnp.float32)
    h1 = jnp.maximum(acc + b1_ref[0][None, :], 0.0)
    out_ref[...] = jnp.sum(h1 * w2_ref[0][None, :], axis=1) + b2_ref[0]


def _tc_mlp(g4, w1r, b1, w2, b2):
    grid = (B // _BM,)
    return pl.pallas_call(
        _mlp_body,
        grid=grid,
        in_specs=[
            pl.BlockSpec((4, _BM, D), lambda i: (0, i, 0)),
            pl.BlockSpec((4, D, D), lambda i: (0, 0, 0)),
            pl.BlockSpec((1, D), lambda i: (0, 0)),
            pl.BlockSpec((1, D), lambda i: (0, 0)),
            pl.BlockSpec((1,), lambda i: (0,)),
        ],
        out_specs=pl.BlockSpec((_BM,), lambda i: (i,)),
        out_shape=jax.ShapeDtypeStruct((B,), jnp.float32),
    )(g4, w1r, b1, w2, b2)


def kernel(src, dst, ts, edge_feat, emb, memory, time_w, time_b, edge_W, edge_b, W1, b1, W2, b2):
    w1r = W1.reshape(D, 4, D).transpose(1, 2, 0)
    g4 = _sc_gather(emb, memory, src, dst)
    return _tc_mlp(g4, w1r, b1.reshape(1, D), W2.reshape(1, D), b2)
